# baseline (speedup 1.0000x reference)
import functools

import jax
import jax.numpy as jnp
from jax import lax
from jax.experimental import pallas as pl
from jax.experimental.pallas import tpu as pltpu
from jax.experimental.pallas import tpu_sc as plsc

N = 50000
E = 800000
E_PAD = 819200
HID = 256
OUT = 512
NG = 8
LANES = 16
WIN = 128
NWIN = E_PAD // WIN
NSC = 2
NTILES = 16
NPAD = 50048
STRIPE = NPAD // NTILES
SPLIT0 = 2816
CF = 64
NCH = HID // CF
BR = 2000
GRID = N // BR


def _sc_seg_sum(table_flat, src3, dst3, zeros, n_chunks, d, dtype):
    mesh = plsc.VectorSubcoreMesh(core_axis_name="c", subcore_axis_name="s")

    @functools.partial(
        pl.kernel,
        out_type=jax.ShapeDtypeStruct((NSC, n_chunks, NPAD, d), dtype),
        mesh=mesh,
        scratch_types=[
            pltpu.VMEM((1, WIN), jnp.int32),
            pltpu.VMEM((1, WIN), jnp.int32),
            pltpu.VMEM((1, WIN), jnp.int32),
            pltpu.VMEM((1, WIN), jnp.int32),
            pltpu.VMEM((WIN, d), dtype),
            pltpu.VMEM((WIN, d), dtype),
            pltpu.VMEM_SHARED((NPAD, d), dtype),
            pltpu.SemaphoreType.DMA,
            pltpu.SemaphoreType.DMA,
            pltpu.SemaphoreType.DMA,
            pltpu.SemaphoreType.DMA,
        ],
        compiler_params=pltpu.CompilerParams(use_tc_tiling_on_sc=False),
    )
    def k(table_hbm, src_hbm, dst_hbm, zero_hbm, out_hbm,
          src_a, dst_a, src_b, dst_b, rows_a, rows_b, acc,
          sem_ia, sem_ib, sem_ga, sem_gb):
        core = lax.axis_index("c")
        sub = lax.axis_index("s")
        r0 = sub * STRIPE
        nwin_c = SPLIT0 + core * (NWIN - 2 * SPLIT0)
        npairs = nwin_c // (2 * NTILES)
        wbase = core * SPLIT0 + sub

        def issue_idx(t, sv, dv, sem):
            w = wbase + t * NTILES
            pltpu.async_copy(src_hbm.at[w], sv, sem)
            pltpu.async_copy(dst_hbm.at[w], dv, sem)

        def wait_idx(sv, dv, sem):
            pltpu.make_async_copy(src_hbm.at[0], sv, sem).wait()
            pltpu.make_async_copy(dst_hbm.at[0], dv, sem).wait()

        def compute_idx(sv, off):
            if n_chunks > 1:
                for kk in range(WIN // LANES):
                    sl = pl.ds(kk * LANES, LANES)
                    sv[0, sl] = sv[0, sl] + off

        def start_gather(sv, rows, sem):
            pltpu.async_copy(table_hbm.at[sv.at[0]], rows, sem)

        def wait_gather(sv, rows, sem):
            pltpu.make_async_copy(table_hbm.at[sv.at[0]], rows, sem).wait()

        def scatter(rows, dv):
            pltpu.sync_copy(rows, acc.at[dv.at[0]], add=True)

        @pl.loop(0, n_chunks)
        def _chunk(cidx):
            off = cidx * N
            pltpu.sync_copy(zero_hbm.at[pl.ds(r0, STRIPE)],
                            acc.at[pl.ds(r0, STRIPE)])
            plsc.subcore_barrier()

            issue_idx(0, src_a, dst_a, sem_ia)
            wait_idx(src_a, dst_a, sem_ia)
            compute_idx(src_a, off)
            start_gather(src_a, rows_a, sem_ga)
            issue_idx(1, src_b, dst_b, sem_ib)

            @pl.loop(0, npairs)
            def _it(k):
                not_last = k < (npairs - 1)
                wait_idx(src_b, dst_b, sem_ib)
                compute_idx(src_b, off)
                wait_gather(src_a, rows_a, sem_ga)
                start_gather(src_b, rows_b, sem_gb)
                scatter(rows_a, dst_a)

                @pl.when(not_last)
                def _():
                    issue_idx(2 * k + 2, src_a, dst_a, sem_ia)
                    wait_idx(src_a, dst_a, sem_ia)
                    compute_idx(src_a, off)

                wait_gather(src_b, rows_b, sem_gb)

                @pl.when(not_last)
                def _():
                    start_gather(src_a, rows_a, sem_ga)
                    issue_idx(2 * k + 3, src_b, dst_b, sem_ib)

                scatter(rows_b, dst_b)

            plsc.subcore_barrier()
            pltpu.sync_copy(acc.at[pl.ds(r0, STRIPE)],
                            out_hbm.at[core, cidx, pl.ds(r0, STRIPE)])

    return k(table_flat, src3, dst3, zeros)


def _tc_layer1(P1, x_pad, wl, wr, b):
    def body(p_ref, x_ref, wl_ref, wr_ref, b_ref, hf_ref, ht_ref, inv_ref):
        s = (p_ref[0].astype(jnp.float32)
             + p_ref[1].astype(jnp.float32))
        cnt = s[:, 3:4]
        inv = 1.0 / jnp.maximum(cnt, 1.0)
        mean = s * inv
        z = (jnp.dot(mean, wl_ref[...], preferred_element_type=jnp.float32)
             + jnp.dot(x_ref[...], wr_ref[...],
                       preferred_element_type=jnp.float32)
             + b_ref[...])
        h = jnp.maximum(z, 0.0)
        hf_ref[...] = h
        for c in range(NCH):
            ht_ref[c] = h[:, c * CF:(c + 1) * CF].astype(jnp.bfloat16)
        inv_ref[...] = inv

    return pl.pallas_call(
        body,
        grid=(GRID,),
        in_specs=[
            pl.BlockSpec((NSC, BR, 32), lambda i: (0, i, 0)),
            pl.BlockSpec((BR, 16), lambda i: (i, 0)),
            pl.BlockSpec((32, HID), lambda i: (0, 0)),
            pl.BlockSpec((16, HID), lambda i: (0, 0)),
            pl.BlockSpec((1, HID), lambda i: (0, 0)),
        ],
        out_specs=[
            pl.BlockSpec((BR, HID), lambda i: (i, 0)),
            pl.BlockSpec((NCH, BR, CF), lambda i: (0, i, 0)),
            pl.BlockSpec((BR, 1), lambda i: (i, 0)),
        ],
        out_shape=[
            jax.ShapeDtypeStruct((N, HID), jnp.float32),
            jax.ShapeDtypeStruct((NCH, N, CF), jnp.bfloat16),
            jax.ShapeDtypeStruct((N, 1), jnp.float32),
        ],
    )(P1, x_pad, wl, wr, b)


def _tc_layer2(P, hf, inv, wl, wr, b):
    def body(p_ref, h_ref, inv_ref, wl_ref, wr_ref, b_ref, hf_ref, ht_ref):
        mean = jnp.concatenate(
            [(p_ref[0, c].astype(jnp.float32)
              + p_ref[1, c].astype(jnp.float32)) for c in range(NCH)], axis=1)
        z = (jnp.dot((mean * inv_ref[...]).astype(jnp.bfloat16), wl_ref[...],
                     preferred_element_type=jnp.float32)
             + jnp.dot(h_ref[...].astype(jnp.bfloat16), wr_ref[...],
                       preferred_element_type=jnp.float32)
             + b_ref[...])
        h = jnp.maximum(z, 0.0)
        hf_ref[...] = h
        for c in range(NCH):
            ht_ref[c] = h[:, c * CF:(c + 1) * CF].astype(jnp.bfloat16)

    return pl.pallas_call(
        body,
        grid=(GRID,),
        in_specs=[
            pl.BlockSpec((NSC, NCH, BR, CF), lambda i: (0, 0, i, 0)),
            pl.BlockSpec((BR, HID), lambda i: (i, 0)),
            pl.BlockSpec((BR, 1), lambda i: (i, 0)),
            pl.BlockSpec((HID, HID), lambda i: (0, 0)),
            pl.BlockSpec((HID, HID), lambda i: (0, 0)),
            pl.BlockSpec((1, HID), lambda i: (0, 0)),
        ],
        out_specs=[
            pl.BlockSpec((BR, HID), lambda i: (i, 0)),
            pl.BlockSpec((NCH, BR, CF), lambda i: (0, i, 0)),
        ],
        out_shape=[
            jax.ShapeDtypeStruct((N, HID), jnp.float32),
            jax.ShapeDtypeStruct((NCH, N, CF), jnp.bfloat16),
        ],
    )(P, hf, inv, wl, wr, b)


def _tc_layer3_pool(P, hf, inv, wl, wr, b, batch2d):
    def body(p_ref, h_ref, inv_ref, wl_ref, wr_ref, b_ref, bat_ref, o_ref,
             psum, pcnt):
        i = pl.program_id(0)

        @pl.when(i == 0)
        def _():
            psum[...] = jnp.zeros_like(psum)
            pcnt[...] = jnp.zeros_like(pcnt)

        mean = jnp.concatenate(
            [(p_ref[0, c].astype(jnp.float32)
              + p_ref[1, c].astype(jnp.float32)) for c in range(NCH)], axis=1)
        z = (jnp.dot((mean * inv_ref[...]).astype(jnp.bfloat16), wl_ref[...],
                     preferred_element_type=jnp.float32)
             + jnp.dot(h_ref[...].astype(jnp.bfloat16), wr_ref[...],
                       preferred_element_type=jnp.float32)
             + b_ref[...])
        onehot = (bat_ref[...] ==
                  lax.broadcasted_iota(jnp.int32, (BR, NG), 1)
                  ).astype(jnp.float32)
        psum[...] += lax.dot_general(
            onehot, z, (((0,), (0,)), ((), ())),
            preferred_element_type=jnp.float32)
        pcnt[...] += lax.dot_general(
            onehot, jnp.ones((BR, 1), jnp.float32),
            (((0,), (0,)), ((), ())),
            preferred_element_type=jnp.float32)

        @pl.when(i == GRID - 1)
        def _():
            o_ref[...] = psum[...] / jnp.maximum(pcnt[...], 1.0)

    return pl.pallas_call(
        body,
        grid=(GRID,),
        in_specs=[
            pl.BlockSpec((NSC, NCH, BR, CF), lambda i: (0, 0, i, 0)),
            pl.BlockSpec((BR, HID), lambda i: (i, 0)),
            pl.BlockSpec((BR, 1), lambda i: (i, 0)),
            pl.BlockSpec((HID, OUT), lambda i: (0, 0)),
            pl.BlockSpec((HID, OUT), lambda i: (0, 0)),
            pl.BlockSpec((1, OUT), lambda i: (0, 0)),
            pl.BlockSpec((BR, 1), lambda i: (i, 0)),
        ],
        out_specs=pl.BlockSpec((NG, OUT), lambda i: (0, 0)),
        out_shape=jax.ShapeDtypeStruct((NG, OUT), jnp.float32),
        scratch_shapes=[
            pltpu.VMEM((NG, OUT), jnp.float32),
            pltpu.VMEM((NG, 1), jnp.float32),
        ],
    )(P, hf, inv, wl, wr, b, batch2d)


def kernel(x, edge_index, batch, W1l, W1r, b1, W2l, W2r, b2, W3l, W3r, b3):
    f32 = jnp.float32
    pad = E_PAD - E
    src3 = jnp.concatenate(
        [edge_index[0].astype(jnp.int32), jnp.zeros((pad,), jnp.int32)]
    ).reshape(NWIN, 1, WIN)
    dst3 = jnp.concatenate(
        [edge_index[1].astype(jnp.int32),
         N + jnp.arange(pad, dtype=jnp.int32) % (NPAD - N)]
    ).reshape(NWIN, 1, WIN)
    batch2d = batch.astype(jnp.int32).reshape(N, 1)

    bf16 = jnp.bfloat16
    x_pad = jnp.concatenate(
        [x.astype(f32), jnp.ones((N, 1), f32), jnp.zeros((N, 12), f32)],
        axis=1)
    x_sc = jnp.concatenate(
        [x.astype(bf16), jnp.ones((N, 1), bf16), jnp.zeros((N, 28), bf16)],
        axis=1)
    z32 = jnp.zeros((NPAD, 32), bf16)
    zcf = jnp.zeros((NPAD, CF), bf16)

    wl1 = jnp.zeros((32, HID), f32).at[:3].set(W1l.T)
    wr1 = jnp.zeros((16, HID), f32).at[:3].set(W1r.T)
    b1r = b1.reshape(1, HID)
    wl2, wr2, b2r = W2l.T.astype(bf16), W2r.T.astype(bf16), b2.reshape(1, HID)
    wl3, wr3, b3r = W3l.T.astype(bf16), W3r.T.astype(bf16), b3.reshape(1, OUT)

    P1 = _sc_seg_sum(x_sc, src3, dst3, z32, 1, 32, bf16)
    h1f, h1t, inv = _tc_layer1(P1.reshape(NSC, NPAD, 32), x_pad,
                               wl1, wr1, b1r)
    P2 = _sc_seg_sum(h1t.reshape(NCH * N, CF), src3, dst3, zcf, NCH, CF,
                     jnp.bfloat16)
    h2f, h2t = _tc_layer2(P2, h1f, inv, wl2, wr2, b2r)
    P3 = _sc_seg_sum(h2t.reshape(NCH * N, CF), src3, dst3, zcf, NCH, CF,
                     jnp.bfloat16)
    return _tc_layer3_pool(P3, h2f, inv, wl3, wr3, b3r, batch2d)


# speedup vs baseline: 1.0653x; 1.0653x over previous
import functools

import jax
import jax.numpy as jnp
from jax import lax
from jax.experimental import pallas as pl
from jax.experimental.pallas import tpu as pltpu
from jax.experimental.pallas import tpu_sc as plsc

N = 50000
E = 800000
E_PAD = 819200
HID = 256
OUT = 512
NG = 8
LANES = 16
WIN = 128
NWIN = E_PAD // WIN
NSC = 2
NTILES = 16
NPAD = 50048
STRIPE = NPAD // NTILES
SPLIT0 = 3584
CF = 64
NCH = HID // CF
BR = 2000
GRID = N // BR


def _sc_seg_sum(table_flat, src3, dst3, zeros, n_chunks, d, dtype):
    mesh = plsc.VectorSubcoreMesh(core_axis_name="c", subcore_axis_name="s")

    @functools.partial(
        pl.kernel,
        out_type=jax.ShapeDtypeStruct((NSC, n_chunks, NPAD, d), dtype),
        mesh=mesh,
        scratch_types=[
            pltpu.VMEM((1, WIN), jnp.int32),
            pltpu.VMEM((1, WIN), jnp.int32),
            pltpu.VMEM((1, WIN), jnp.int32),
            pltpu.VMEM((1, WIN), jnp.int32),
            pltpu.VMEM((WIN, d), dtype),
            pltpu.VMEM((WIN, d), dtype),
            pltpu.VMEM_SHARED((NPAD, d), dtype),
            pltpu.SemaphoreType.DMA,
            pltpu.SemaphoreType.DMA,
            pltpu.SemaphoreType.DMA,
            pltpu.SemaphoreType.DMA,
        ],
        compiler_params=pltpu.CompilerParams(use_tc_tiling_on_sc=False),
    )
    def k(table_hbm, src_hbm, dst_hbm, zero_hbm, out_hbm,
          src_a, dst_a, src_b, dst_b, rows_a, rows_b, acc,
          sem_ia, sem_ib, sem_ga, sem_gb):
        core = lax.axis_index("c")
        sub = lax.axis_index("s")
        r0 = sub * STRIPE
        nwin_c = SPLIT0 + core * (NWIN - 2 * SPLIT0)
        npairs = nwin_c // (2 * NTILES)
        wbase = core * SPLIT0 + sub

        def issue_idx(t, sv, dv, sem):
            w = wbase + t * NTILES
            pltpu.async_copy(src_hbm.at[w], sv, sem)
            pltpu.async_copy(dst_hbm.at[w], dv, sem)

        def wait_idx(sv, dv, sem):
            pltpu.make_async_copy(src_hbm.at[0], sv, sem).wait()
            pltpu.make_async_copy(dst_hbm.at[0], dv, sem).wait()

        def compute_idx(sv, off):
            if n_chunks > 1:
                for kk in range(WIN // LANES):
                    sl = pl.ds(kk * LANES, LANES)
                    sv[0, sl] = sv[0, sl] + off

        def start_gather(sv, rows, sem):
            pltpu.async_copy(table_hbm.at[sv.at[0]], rows, sem)

        def wait_gather(sv, rows, sem):
            pltpu.make_async_copy(table_hbm.at[sv.at[0]], rows, sem).wait()

        def scatter(rows, dv):
            pltpu.sync_copy(rows, acc.at[dv.at[0]], add=True)

        @pl.loop(0, n_chunks)
        def _chunk(cidx):
            off = cidx * N
            pltpu.sync_copy(zero_hbm.at[pl.ds(r0, STRIPE)],
                            acc.at[pl.ds(r0, STRIPE)])
            plsc.subcore_barrier()

            issue_idx(0, src_a, dst_a, sem_ia)
            wait_idx(src_a, dst_a, sem_ia)
            compute_idx(src_a, off)
            start_gather(src_a, rows_a, sem_ga)
            issue_idx(1, src_b, dst_b, sem_ib)

            @pl.loop(0, npairs)
            def _it(k):
                not_last = k < (npairs - 1)
                wait_idx(src_b, dst_b, sem_ib)
                compute_idx(src_b, off)
                wait_gather(src_a, rows_a, sem_ga)
                start_gather(src_b, rows_b, sem_gb)
                scatter(rows_a, dst_a)

                @pl.when(not_last)
                def _():
                    issue_idx(2 * k + 2, src_a, dst_a, sem_ia)
                    wait_idx(src_a, dst_a, sem_ia)
                    compute_idx(src_a, off)

                wait_gather(src_b, rows_b, sem_gb)

                @pl.when(not_last)
                def _():
                    start_gather(src_a, rows_a, sem_ga)
                    issue_idx(2 * k + 3, src_b, dst_b, sem_ib)

                scatter(rows_b, dst_b)

            plsc.subcore_barrier()
            pltpu.sync_copy(acc.at[pl.ds(r0, STRIPE)],
                            out_hbm.at[core, cidx, pl.ds(r0, STRIPE)])

    return k(table_flat, src3, dst3, zeros)


def _tc_layer1(P1, x_pad, wl, wr, b):
    def body(p_ref, x_ref, wl_ref, wr_ref, b_ref, hf_ref, ht_ref, inv_ref):
        s = (p_ref[0].astype(jnp.float32)
             + p_ref[1].astype(jnp.float32))
        cnt = s[:, 3:4]
        inv = 1.0 / jnp.maximum(cnt, 1.0)
        mean = s * inv
        z = (jnp.dot(mean, wl_ref[...], preferred_element_type=jnp.float32)
             + jnp.dot(x_ref[...], wr_ref[...],
                       preferred_element_type=jnp.float32)
             + b_ref[...])
        h = jnp.maximum(z, 0.0)
        hf_ref[...] = h
        for c in range(NCH):
            ht_ref[c] = h[:, c * CF:(c + 1) * CF].astype(jnp.bfloat16)
        inv_ref[...] = inv

    return pl.pallas_call(
        body,
        grid=(GRID,),
        in_specs=[
            pl.BlockSpec((NSC, BR, 32), lambda i: (0, i, 0)),
            pl.BlockSpec((BR, 16), lambda i: (i, 0)),
            pl.BlockSpec((32, HID), lambda i: (0, 0)),
            pl.BlockSpec((16, HID), lambda i: (0, 0)),
            pl.BlockSpec((1, HID), lambda i: (0, 0)),
        ],
        out_specs=[
            pl.BlockSpec((BR, HID), lambda i: (i, 0)),
            pl.BlockSpec((NCH, BR, CF), lambda i: (0, i, 0)),
            pl.BlockSpec((BR, 1), lambda i: (i, 0)),
        ],
        out_shape=[
            jax.ShapeDtypeStruct((N, HID), jnp.float32),
            jax.ShapeDtypeStruct((NCH, N, CF), jnp.bfloat16),
            jax.ShapeDtypeStruct((N, 1), jnp.float32),
        ],
    )(P1, x_pad, wl, wr, b)


def _tc_layer2(P, hf, inv, wl, wr, b):
    def body(p_ref, h_ref, inv_ref, wl_ref, wr_ref, b_ref, hf_ref, ht_ref):
        mean = jnp.concatenate(
            [(p_ref[0, c].astype(jnp.float32)
              + p_ref[1, c].astype(jnp.float32)) for c in range(NCH)], axis=1)
        z = (jnp.dot((mean * inv_ref[...]).astype(jnp.bfloat16), wl_ref[...],
                     preferred_element_type=jnp.float32)
             + jnp.dot(h_ref[...].astype(jnp.bfloat16), wr_ref[...],
                       preferred_element_type=jnp.float32)
             + b_ref[...])
        h = jnp.maximum(z, 0.0)
        hf_ref[...] = h
        for c in range(NCH):
            ht_ref[c] = h[:, c * CF:(c + 1) * CF].astype(jnp.bfloat16)

    return pl.pallas_call(
        body,
        grid=(GRID,),
        in_specs=[
            pl.BlockSpec((NSC, NCH, BR, CF), lambda i: (0, 0, i, 0)),
            pl.BlockSpec((BR, HID), lambda i: (i, 0)),
            pl.BlockSpec((BR, 1), lambda i: (i, 0)),
            pl.BlockSpec((HID, HID), lambda i: (0, 0)),
            pl.BlockSpec((HID, HID), lambda i: (0, 0)),
            pl.BlockSpec((1, HID), lambda i: (0, 0)),
        ],
        out_specs=[
            pl.BlockSpec((BR, HID), lambda i: (i, 0)),
            pl.BlockSpec((NCH, BR, CF), lambda i: (0, i, 0)),
        ],
        out_shape=[
            jax.ShapeDtypeStruct((N, HID), jnp.float32),
            jax.ShapeDtypeStruct((NCH, N, CF), jnp.bfloat16),
        ],
    )(P, hf, inv, wl, wr, b)


def _tc_layer3_pool(P, hf, inv, wl, wr, b, batch2d):
    def body(p_ref, h_ref, inv_ref, wl_ref, wr_ref, b_ref, bat_ref, o_ref,
             psum, pcnt):
        i = pl.program_id(0)

        @pl.when(i == 0)
        def _():
            psum[...] = jnp.zeros_like(psum)
            pcnt[...] = jnp.zeros_like(pcnt)

        mean = jnp.concatenate(
            [(p_ref[0, c].astype(jnp.float32)
              + p_ref[1, c].astype(jnp.float32)) for c in range(NCH)], axis=1)
        z = (jnp.dot((mean * inv_ref[...]).astype(jnp.bfloat16), wl_ref[...],
                     preferred_element_type=jnp.float32)
             + jnp.dot(h_ref[...].astype(jnp.bfloat16), wr_ref[...],
                       preferred_element_type=jnp.float32)
             + b_ref[...])
        onehot = (bat_ref[...] ==
                  lax.broadcasted_iota(jnp.int32, (BR, NG), 1)
                  ).astype(jnp.float32)
        psum[...] += lax.dot_general(
            onehot, z, (((0,), (0,)), ((), ())),
            preferred_element_type=jnp.float32)
        pcnt[...] += lax.dot_general(
            onehot, jnp.ones((BR, 1), jnp.float32),
            (((0,), (0,)), ((), ())),
            preferred_element_type=jnp.float32)

        @pl.when(i == GRID - 1)
        def _():
            o_ref[...] = psum[...] / jnp.maximum(pcnt[...], 1.0)

    return pl.pallas_call(
        body,
        grid=(GRID,),
        in_specs=[
            pl.BlockSpec((NSC, NCH, BR, CF), lambda i: (0, 0, i, 0)),
            pl.BlockSpec((BR, HID), lambda i: (i, 0)),
            pl.BlockSpec((BR, 1), lambda i: (i, 0)),
            pl.BlockSpec((HID, OUT), lambda i: (0, 0)),
            pl.BlockSpec((HID, OUT), lambda i: (0, 0)),
            pl.BlockSpec((1, OUT), lambda i: (0, 0)),
            pl.BlockSpec((BR, 1), lambda i: (i, 0)),
        ],
        out_specs=pl.BlockSpec((NG, OUT), lambda i: (0, 0)),
        out_shape=jax.ShapeDtypeStruct((NG, OUT), jnp.float32),
        scratch_shapes=[
            pltpu.VMEM((NG, OUT), jnp.float32),
            pltpu.VMEM((NG, 1), jnp.float32),
        ],
    )(P, hf, inv, wl, wr, b, batch2d)


def kernel(x, edge_index, batch, W1l, W1r, b1, W2l, W2r, b2, W3l, W3r, b3):
    f32 = jnp.float32
    pad = E_PAD - E
    src3 = jnp.concatenate(
        [edge_index[0].astype(jnp.int32), jnp.zeros((pad,), jnp.int32)]
    ).reshape(NWIN, 1, WIN)
    dst3 = jnp.concatenate(
        [edge_index[1].astype(jnp.int32),
         N + jnp.arange(pad, dtype=jnp.int32) % (NPAD - N)]
    ).reshape(NWIN, 1, WIN)
    batch2d = batch.astype(jnp.int32).reshape(N, 1)

    bf16 = jnp.bfloat16
    x_pad = jnp.concatenate(
        [x.astype(f32), jnp.ones((N, 1), f32), jnp.zeros((N, 12), f32)],
        axis=1)
    x_sc = jnp.concatenate(
        [x.astype(bf16), jnp.ones((N, 1), bf16), jnp.zeros((N, 28), bf16)],
        axis=1)
    z32 = jnp.zeros((NPAD, 32), bf16)
    zcf = jnp.zeros((NPAD, CF), bf16)

    wl1 = jnp.zeros((32, HID), f32).at[:3].set(W1l.T)
    wr1 = jnp.zeros((16, HID), f32).at[:3].set(W1r.T)
    b1r = b1.reshape(1, HID)
    wl2, wr2, b2r = W2l.T.astype(bf16), W2r.T.astype(bf16), b2.reshape(1, HID)
    wl3, wr3, b3r = W3l.T.astype(bf16), W3r.T.astype(bf16), b3.reshape(1, OUT)

    P1 = _sc_seg_sum(x_sc, src3, dst3, z32, 1, 32, bf16)
    h1f, h1t, inv = _tc_layer1(P1.reshape(NSC, NPAD, 32), x_pad,
                               wl1, wr1, b1r)
    P2 = _sc_seg_sum(h1t.reshape(NCH * N, CF), src3, dst3, zcf, NCH, CF,
                     jnp.bfloat16)
    h2f, h2t = _tc_layer2(P2, h1f, inv, wl2, wr2, b2r)
    P3 = _sc_seg_sum(h2t.reshape(NCH * N, CF), src3, dst3, zcf, NCH, CF,
                     jnp.bfloat16)
    return _tc_layer3_pool(P3, h2f, inv, wl3, wr3, b3r, batch2d)
